# Initial kernel scaffold; baseline (speedup 1.0000x reference)
import jax
import jax.numpy as jnp
from jax.experimental import pallas as pl


def kernel(logits, uniform):
    raise NotImplementedError("write your pallas kernel here")


# speedup vs baseline: 10.4525x; 10.4525x over previous
import functools

import jax
import jax.numpy as jnp
from jax.experimental import pallas as pl
from jax.experimental.pallas import tpu as pltpu

_TAU = 0.5
_K = 8
_ROWS = 8


def _body(logits_ref, u_ref, w_ref, sel_ref):
    lg = logits_ref[...]
    rows, d = lg.shape
    m = jnp.max(lg, axis=-1, keepdims=True)
    g = jnp.exp((lg - m) * (1.0 / _TAU))

    w = jnp.zeros_like(lg)
    for k in range(_K):
        u = jnp.clip(u_ref[:, k, :], 0.0001, 0.9999)
        lnu = jnp.log(u)
        e = g / (lnu * lnu)
        s = jnp.sum(e, axis=-1, keepdims=True)
        w = jnp.maximum(w, e * (1.0 / s))
    w_ref[...] = w

    lane = jax.lax.broadcasted_iota(jnp.int32, (rows, d), 1)
    x = lg
    for _ in range(_K - 1):
        mx = jnp.max(x, axis=-1, keepdims=True)
        idx = jnp.min(jnp.where(x == mx, lane, d), axis=-1, keepdims=True)
        x = jnp.where(lane == idx, -jnp.inf, x)
    thresh = jnp.max(x, axis=-1, keepdims=True)
    sel_ref[...] = (lg >= thresh).astype(jnp.float32)


@functools.partial(jax.jit, static_argnames=())
def kernel(logits, uniform):
    b, d = logits.shape
    k = uniform.shape[1]
    grid = (b // _ROWS,)
    w, sel = pl.pallas_call(
        _body,
        grid=grid,
        in_specs=[
            pl.BlockSpec((_ROWS, d), lambda i: (i, 0)),
            pl.BlockSpec((_ROWS, k, d), lambda i: (i, 0, 0)),
        ],
        out_specs=[
            pl.BlockSpec((_ROWS, d), lambda i: (i, 0)),
            pl.BlockSpec((_ROWS, d), lambda i: (i, 0)),
        ],
        out_shape=[
            jax.ShapeDtypeStruct((b, d), jnp.float32),
            jax.ShapeDtypeStruct((b, d), jnp.float32),
        ],
        compiler_params=pltpu.CompilerParams(
            dimension_semantics=("arbitrary",),
        ),
    )(logits, uniform)
    return (w, sel)
